# baseline (speedup 1.0000x reference)
import functools

import jax
import jax.numpy as jnp
from jax.experimental import pallas as pl
from jax.experimental.pallas import tpu as pltpu

_B = 16
_T = 2048
_CB = 1024
_D = 32
_TB = 512
_HB = _TB // 2
_TPB = _T // _TB
_N = _B * _T
_BETA = 0.25


def _vq_body(cb2_ref, x_ref, cbm2_ref, cbT_ref, ones_ref, irow_ref, icol_ref,
             out_ref, loss_ref, perp_ref, acc_ref, cnt_ref):
    step = pl.program_id(0)
    nsteps = pl.num_programs(0)

    @pl.when(step == 0)
    def _init():
        acc_ref[0] = 0.0
        cnt_ref[...] = jnp.zeros_like(cnt_ref)

    cb2 = cb2_ref[...]

    def _half(lo):
        xh = jax.lax.slice(x_ref[0], (0, lo), (_D, lo + _HB))
        xb = jnp.transpose(xh, (1, 0))
        sx = jnp.sum(xb * xb, axis=1, keepdims=True)
        mm2 = jax.lax.dot_general(
            xb, cbm2_ref[...], dimension_numbers=(((1,), (1,)), ((), ())),
            preferred_element_type=jnp.float32)
        dist = (sx + cb2) + mm2

        mn = jnp.min(dist, axis=1, keepdims=True)
        idx = jnp.min(jnp.where(dist == mn, irow_ref[...], 2048.0),
                      axis=1, keepdims=True)
        onehot = (irow_ref[...] == idx).astype(jnp.float32)

        idxT = jnp.transpose(idx, (1, 0))
        onehotT = (icol_ref[...] == idxT).astype(jnp.float32)
        qT = jax.lax.dot_general(
            cbT_ref[...], onehotT, dimension_numbers=(((1,), (0,)), ((), ())),
            preferred_element_type=jnp.float32)
        return qT, onehot, mn

    qT0, oh0, mn0 = _half(0)
    qT1, oh1, mn1 = _half(_HB)
    out_ref[0] = jnp.concatenate([qT0, qT1], axis=1)

    oness = ones_ref[...]
    cnt_ref[...] += jax.lax.dot_general(
        oness, oh0, dimension_numbers=(((1,), (0,)), ((), ())),
        preferred_element_type=jnp.float32)
    cnt_ref[...] += jax.lax.dot_general(
        oness, oh1, dimension_numbers=(((1,), (0,)), ((), ())),
        preferred_element_type=jnp.float32)

    acc_ref[0] += jnp.sum(mn0) + jnp.sum(mn1)

    @pl.when(step == nsteps - 1)
    def _fin():
        m = acc_ref[0] * (1.0 / (_N * _D))
        loss_ref[0, 0] = m + _BETA * m
        p = cnt_ref[...] * (1.0 / _N)
        perp_ref[0, 0] = jnp.exp(-jnp.sum(p * jnp.log(p + 1e-10)))


@functools.partial(jax.jit, static_argnames=("interpret",))
def _vq_call(x, cb2, cbm2, cb, ones, irow, icol, interpret=False):
    nsteps = _N // _TB
    content, loss, perp = pl.pallas_call(
        _vq_body,
        grid=(nsteps,),
        in_specs=[
            pl.BlockSpec((1, _CB), lambda i: (0, 0)),
            pl.BlockSpec((1, _D, _TB), lambda i: (i // _TPB, 0, i % _TPB)),
            pl.BlockSpec((_CB, _D), lambda i: (0, 0)),
            pl.BlockSpec((_D, _CB), lambda i: (0, 0)),
            pl.BlockSpec((1, _HB), lambda i: (0, 0)),
            pl.BlockSpec((1, _CB), lambda i: (0, 0)),
            pl.BlockSpec((_CB, 1), lambda i: (0, 0)),
        ],
        out_specs=[
            pl.BlockSpec((1, _D, _TB), lambda i: (i // _TPB, 0, i % _TPB)),
            pl.BlockSpec(memory_space=pltpu.SMEM),
            pl.BlockSpec(memory_space=pltpu.SMEM),
        ],
        out_shape=[
            jax.ShapeDtypeStruct((_B, _D, _T), jnp.float32),
            jax.ShapeDtypeStruct((1, 1), jnp.float32),
            jax.ShapeDtypeStruct((1, 1), jnp.float32),
        ],
        scratch_shapes=[
            pltpu.SMEM((1,), jnp.float32),
            pltpu.VMEM((1, _CB), jnp.float32),
        ],
        interpret=interpret,
    )(cb2, x, cbm2, cb, ones, irow, icol)
    return content, loss, perp


def kernel(x, codebook):
    cb2 = jnp.sum(codebook ** 2, axis=1)[None, :]
    ones = jnp.ones((1, _HB), jnp.float32)
    irow = jnp.arange(_CB, dtype=jnp.float32)[None, :]
    icol = jnp.arange(_CB, dtype=jnp.float32)[:, None]
    content, loss, perp = _vq_call(x, cb2, codebook * (-2.0), codebook.T, ones,
                                   irow, icol)
    return content, loss.reshape(()), perp.reshape(())


# speedup vs baseline: 1.4845x; 1.4845x over previous
import functools

import jax
import jax.numpy as jnp
from jax.experimental import pallas as pl
from jax.experimental.pallas import tpu as pltpu

_B = 16
_T = 2048
_CB = 1024
_D = 32
_TB = 1024
_TPB = _T // _TB
_N = _B * _T
_BETA = 0.25


def _vq_body(cb2_ref, x_ref, cbm2_ref, cbT_ref, ones_ref, irow_ref, icol_ref,
             out_ref, loss_ref, perp_ref, acc_ref, cnt_ref):
    step = pl.program_id(0)
    nsteps = pl.num_programs(0)

    @pl.when(step == 0)
    def _init():
        acc_ref[0] = 0.0
        cnt_ref[...] = jnp.zeros_like(cnt_ref)

    xb = jnp.transpose(x_ref[0], (1, 0))
    sx = jnp.sum(xb * xb, axis=1, keepdims=True)
    cb2 = cb2_ref[...]

    mm2 = jax.lax.dot_general(
        xb, cbm2_ref[...], dimension_numbers=(((1,), (1,)), ((), ())),
        preferred_element_type=jnp.float32)
    dist = (sx + cb2) + mm2

    mn = jnp.min(dist, axis=1, keepdims=True)
    idx = jnp.min(jnp.where(dist == mn, irow_ref[...], 2048.0),
                  axis=1, keepdims=True)
    onehot = (irow_ref[...] == idx).astype(jnp.float32)

    idxT = jnp.transpose(idx, (1, 0))
    onehotT = (icol_ref[...] == idxT).astype(jnp.float32)
    out_ref[0] = jax.lax.dot_general(
        cbT_ref[...], onehotT, dimension_numbers=(((1,), (0,)), ((), ())),
        preferred_element_type=jnp.float32)

    cnt_ref[...] += jax.lax.dot_general(
        ones_ref[...], onehot, dimension_numbers=(((1,), (0,)), ((), ())),
        preferred_element_type=jnp.float32)

    acc_ref[0] += jnp.sum(mn)

    @pl.when(step == nsteps - 1)
    def _fin():
        m = acc_ref[0] * (1.0 / (_N * _D))
        loss_ref[0, 0] = m + _BETA * m
        p = cnt_ref[...] * (1.0 / _N)
        perp_ref[0, 0] = jnp.exp(-jnp.sum(p * jnp.log(p + 1e-10)))


@functools.partial(jax.jit, static_argnames=("interpret",))
def _vq_call(x, cb2, cbm2, cb, ones, irow, icol, interpret=False):
    nsteps = _N // _TB
    content, loss, perp = pl.pallas_call(
        _vq_body,
        grid=(nsteps,),
        in_specs=[
            pl.BlockSpec((1, _CB), lambda i: (0, 0)),
            pl.BlockSpec((1, _D, _TB), lambda i: (i // _TPB, 0, i % _TPB)),
            pl.BlockSpec((_CB, _D), lambda i: (0, 0)),
            pl.BlockSpec((_D, _CB), lambda i: (0, 0)),
            pl.BlockSpec((1, _TB), lambda i: (0, 0)),
            pl.BlockSpec((1, _CB), lambda i: (0, 0)),
            pl.BlockSpec((_CB, 1), lambda i: (0, 0)),
        ],
        out_specs=[
            pl.BlockSpec((1, _D, _TB), lambda i: (i // _TPB, 0, i % _TPB)),
            pl.BlockSpec(memory_space=pltpu.SMEM),
            pl.BlockSpec(memory_space=pltpu.SMEM),
        ],
        out_shape=[
            jax.ShapeDtypeStruct((_B, _D, _T), jnp.float32),
            jax.ShapeDtypeStruct((1, 1), jnp.float32),
            jax.ShapeDtypeStruct((1, 1), jnp.float32),
        ],
        scratch_shapes=[
            pltpu.SMEM((1,), jnp.float32),
            pltpu.VMEM((1, _CB), jnp.float32),
        ],
        interpret=interpret,
    )(cb2, x, cbm2, cb, ones, irow, icol)
    return content, loss, perp


def kernel(x, codebook):
    cb2 = jnp.sum(codebook ** 2, axis=1)[None, :]
    ones = jnp.ones((1, _TB), jnp.float32)
    irow = jnp.arange(_CB, dtype=jnp.float32)[None, :]
    icol = jnp.arange(_CB, dtype=jnp.float32)[:, None]
    content, loss, perp = _vq_call(x, cb2, codebook * (-2.0), codebook.T, ones,
                                   irow, icol)
    return content, loss.reshape(()), perp.reshape(())


# speedup vs baseline: 1.6497x; 1.1113x over previous
import functools

import jax
import jax.numpy as jnp
from jax.experimental import pallas as pl
from jax.experimental.pallas import tpu as pltpu

_B = 16
_T = 2048
_CB = 1024
_D = 32
_TB = 2048
_TPB = _T // _TB
_N = _B * _T
_BETA = 0.25


def _vq_body(cb2_ref, x_ref, cbm2_ref, cbT_ref, ones_ref, irow_ref, icol_ref,
             out_ref, loss_ref, perp_ref, acc_ref, cnt_ref):
    step = pl.program_id(0)
    nsteps = pl.num_programs(0)

    @pl.when(step == 0)
    def _init():
        acc_ref[0] = 0.0
        cnt_ref[...] = jnp.zeros_like(cnt_ref)

    xb = jnp.transpose(x_ref[0], (1, 0))
    sx = jnp.sum(xb * xb, axis=1, keepdims=True)
    cb2 = cb2_ref[...]

    mm2 = jax.lax.dot_general(
        xb, cbm2_ref[...], dimension_numbers=(((1,), (1,)), ((), ())),
        preferred_element_type=jnp.float32)
    dist = (sx + cb2) + mm2

    mn = jnp.min(dist, axis=1, keepdims=True)
    idx = jnp.min(jnp.where(dist == mn, irow_ref[...], 2048.0),
                  axis=1, keepdims=True)
    onehot = (irow_ref[...] == idx).astype(jnp.float32)

    idxT = jnp.transpose(idx, (1, 0))
    onehotT = (icol_ref[...] == idxT).astype(jnp.float32)
    out_ref[0] = jax.lax.dot_general(
        cbT_ref[...], onehotT, dimension_numbers=(((1,), (0,)), ((), ())),
        preferred_element_type=jnp.float32)

    cnt_ref[...] += jax.lax.dot_general(
        ones_ref[...], onehot, dimension_numbers=(((1,), (0,)), ((), ())),
        preferred_element_type=jnp.float32)

    acc_ref[0] += jnp.sum(mn)

    @pl.when(step == nsteps - 1)
    def _fin():
        m = acc_ref[0] * (1.0 / (_N * _D))
        loss_ref[0, 0] = m + _BETA * m
        p = cnt_ref[...] * (1.0 / _N)
        perp_ref[0, 0] = jnp.exp(-jnp.sum(p * jnp.log(p + 1e-10)))


@functools.partial(jax.jit, static_argnames=("interpret",))
def _vq_call(x, cb2, cbm2, cb, ones, irow, icol, interpret=False):
    nsteps = _N // _TB
    content, loss, perp = pl.pallas_call(
        _vq_body,
        grid=(nsteps,),
        in_specs=[
            pl.BlockSpec((1, _CB), lambda i: (0, 0)),
            pl.BlockSpec((1, _D, _TB), lambda i: (i // _TPB, 0, i % _TPB)),
            pl.BlockSpec((_CB, _D), lambda i: (0, 0)),
            pl.BlockSpec((_D, _CB), lambda i: (0, 0)),
            pl.BlockSpec((1, _TB), lambda i: (0, 0)),
            pl.BlockSpec((1, _CB), lambda i: (0, 0)),
            pl.BlockSpec((_CB, 1), lambda i: (0, 0)),
        ],
        out_specs=[
            pl.BlockSpec((1, _D, _TB), lambda i: (i // _TPB, 0, i % _TPB)),
            pl.BlockSpec(memory_space=pltpu.SMEM),
            pl.BlockSpec(memory_space=pltpu.SMEM),
        ],
        out_shape=[
            jax.ShapeDtypeStruct((_B, _D, _T), jnp.float32),
            jax.ShapeDtypeStruct((1, 1), jnp.float32),
            jax.ShapeDtypeStruct((1, 1), jnp.float32),
        ],
        scratch_shapes=[
            pltpu.SMEM((1,), jnp.float32),
            pltpu.VMEM((1, _CB), jnp.float32),
        ],
        interpret=interpret,
    )(cb2, x, cbm2, cb, ones, irow, icol)
    return content, loss, perp


def kernel(x, codebook):
    cb2 = jnp.sum(codebook ** 2, axis=1)[None, :]
    ones = jnp.ones((1, _TB), jnp.float32)
    irow = jnp.arange(_CB, dtype=jnp.float32)[None, :]
    icol = jnp.arange(_CB, dtype=jnp.float32)[:, None]
    content, loss, perp = _vq_call(x, cb2, codebook * (-2.0), codebook.T, ones,
                                   irow, icol)
    return content, loss.reshape(()), perp.reshape(())


# speedup vs baseline: 1.6508x; 1.0007x over previous
import functools

import jax
import jax.numpy as jnp
from jax.experimental import pallas as pl
from jax.experimental.pallas import tpu as pltpu

_B = 16
_T = 2048
_CB = 1024
_D = 32
_TB = 2048
_RPS = 2
_N = _B * _T
_BETA = 0.25


def _vq_body(cb2_ref, x_ref, cbm2_ref, cbT_ref, ones_ref, irow_ref, icol_ref,
             out_ref, loss_ref, perp_ref, acc_ref, cnt_ref):
    step = pl.program_id(0)
    nsteps = pl.num_programs(0)

    @pl.when(step == 0)
    def _init():
        acc_ref[0] = 0.0
        cnt_ref[...] = jnp.zeros_like(cnt_ref)

    cb2 = cb2_ref[...]

    for r in range(_RPS):
        xb = jnp.transpose(x_ref[r], (1, 0))
        sx = jnp.sum(xb * xb, axis=1, keepdims=True)

        mm2 = jax.lax.dot_general(
            xb, cbm2_ref[...], dimension_numbers=(((1,), (1,)), ((), ())),
            preferred_element_type=jnp.float32)
        dist = (sx + cb2) + mm2

        mn = jnp.min(dist, axis=1, keepdims=True)
        idx = jnp.min(jnp.where(dist == mn, irow_ref[...], 2048.0),
                      axis=1, keepdims=True)
        onehot = (irow_ref[...] == idx).astype(jnp.float32)

        idxT = jnp.transpose(idx, (1, 0))
        onehotT = (icol_ref[...] == idxT).astype(jnp.float32)
        out_ref[r] = jax.lax.dot_general(
            cbT_ref[...], onehotT, dimension_numbers=(((1,), (0,)), ((), ())),
            preferred_element_type=jnp.float32)

        cnt_ref[...] += jax.lax.dot_general(
            ones_ref[...], onehot, dimension_numbers=(((1,), (0,)), ((), ())),
            preferred_element_type=jnp.float32)

        acc_ref[0] += jnp.sum(mn)

    @pl.when(step == nsteps - 1)
    def _fin():
        m = acc_ref[0] * (1.0 / (_N * _D))
        loss_ref[0, 0] = m + _BETA * m
        p = cnt_ref[...] * (1.0 / _N)
        perp_ref[0, 0] = jnp.exp(-jnp.sum(p * jnp.log(p + 1e-10)))


@functools.partial(jax.jit, static_argnames=("interpret",))
def _vq_call(x, cb2, cbm2, cb, ones, irow, icol, interpret=False):
    nsteps = _B // _RPS
    content, loss, perp = pl.pallas_call(
        _vq_body,
        grid=(nsteps,),
        in_specs=[
            pl.BlockSpec((1, _CB), lambda i: (0, 0)),
            pl.BlockSpec((_RPS, _D, _TB), lambda i: (i, 0, 0)),
            pl.BlockSpec((_CB, _D), lambda i: (0, 0)),
            pl.BlockSpec((_D, _CB), lambda i: (0, 0)),
            pl.BlockSpec((1, _TB), lambda i: (0, 0)),
            pl.BlockSpec((1, _CB), lambda i: (0, 0)),
            pl.BlockSpec((_CB, 1), lambda i: (0, 0)),
        ],
        out_specs=[
            pl.BlockSpec((_RPS, _D, _TB), lambda i: (i, 0, 0)),
            pl.BlockSpec(memory_space=pltpu.SMEM),
            pl.BlockSpec(memory_space=pltpu.SMEM),
        ],
        out_shape=[
            jax.ShapeDtypeStruct((_B, _D, _T), jnp.float32),
            jax.ShapeDtypeStruct((1, 1), jnp.float32),
            jax.ShapeDtypeStruct((1, 1), jnp.float32),
        ],
        scratch_shapes=[
            pltpu.SMEM((1,), jnp.float32),
            pltpu.VMEM((1, _CB), jnp.float32),
        ],
        interpret=interpret,
    )(cb2, x, cbm2, cb, ones, irow, icol)
    return content, loss, perp


def kernel(x, codebook):
    cb2 = jnp.sum(codebook ** 2, axis=1)[None, :]
    ones = jnp.ones((1, _TB), jnp.float32)
    irow = jnp.arange(_CB, dtype=jnp.float32)[None, :]
    icol = jnp.arange(_CB, dtype=jnp.float32)[:, None]
    content, loss, perp = _vq_call(x, cb2, codebook * (-2.0), codebook.T, ones,
                                   irow, icol)
    return content, loss.reshape(()), perp.reshape(())
